# baseline (speedup 1.0000x reference)
import jax
import jax.numpy as jnp
from jax import lax
from jax.experimental import pallas as pl
from jax.experimental.pallas import tpu as pltpu

TOKENS = 2048
HIDDEN = 1024
E = 8
TOPK = 2
NGROUP = 4
EG = E // NGROUP
DFF = 512
SHARED_INTER = 1024
RSF = 2.5

TB = 256


def _relu2(x):
    return jnp.square(jnp.maximum(x, 0.0))


def _moe_block(x_ref, gw_ref, bias_ref, w1_ref, w2_ref, sw1_ref, sw2_ref, out_ref):
    x = x_ref[...]

    logits = jnp.dot(x, gw_ref[...].T, preferred_element_type=jnp.float32)
    scores = jax.nn.sigmoid(logits)
    swb = scores + bias_ref[...]

    gs = swb.reshape(TB, NGROUP, EG).sum(axis=-1)
    gidx = lax.broadcasted_iota(jnp.int32, (TB, NGROUP), 1)
    g1 = jnp.argmax(gs, axis=1)
    gs2 = jnp.where(gidx == g1[:, None], -jnp.inf, gs)
    g2 = jnp.argmax(gs2, axis=1)

    eidx = lax.broadcasted_iota(jnp.int32, (TB, E), 1)
    egrp = eidx // EG
    emask = (egrp == g1[:, None]) | (egrp == g2[:, None])
    masked = jnp.where(emask, swb, -jnp.inf)
    e1 = jnp.argmax(masked, axis=1)
    m2 = jnp.where(eidx == e1[:, None], -jnp.inf, masked)
    e2 = jnp.argmax(m2, axis=1)
    oh1 = (eidx == e1[:, None]).astype(jnp.float32)
    oh2 = (eidx == e2[:, None]).astype(jnp.float32)
    s1 = jnp.sum(oh1 * scores, axis=1)
    s2 = jnp.sum(oh2 * scores, axis=1)
    rn = RSF / (s1 + s2 + 1e-20)
    gates = oh1 * (s1 * rn)[:, None] + oh2 * (s2 * rn)[:, None]

    xb = x.astype(jnp.bfloat16)

    h = _relu2(jnp.dot(xb, sw1_ref[...], preferred_element_type=jnp.float32))
    acc = jnp.dot(h.astype(jnp.bfloat16), sw2_ref[...],
                  preferred_element_type=jnp.float32)

    for e in range(E):
        he = _relu2(jnp.dot(xb, w1_ref[e], preferred_element_type=jnp.float32))
        ye = jnp.dot(he.astype(jnp.bfloat16), w2_ref[e],
                     preferred_element_type=jnp.float32)
        acc = acc + gates[:, e:e + 1] * ye

    out_ref[...] = acc


def kernel(hidden_states, gate_weight, e_score_correction_bias, w1, w2, shared_w1, shared_w2):
    orig_shape = hidden_states.shape
    x = hidden_states.reshape(-1, HIDDEN)
    w1b = w1.astype(jnp.bfloat16)
    w2b = w2.astype(jnp.bfloat16)
    sw1b = shared_w1.astype(jnp.bfloat16)
    sw2b = shared_w2.astype(jnp.bfloat16)

    grid = (TOKENS // TB,)
    out = pl.pallas_call(
        _moe_block,
        grid=grid,
        in_specs=[
            pl.BlockSpec((TB, HIDDEN), lambda i: (i, 0)),
            pl.BlockSpec((E, HIDDEN), lambda i: (0, 0)),
            pl.BlockSpec((E,), lambda i: (0,)),
            pl.BlockSpec((E, HIDDEN, DFF), lambda i: (0, 0, 0)),
            pl.BlockSpec((E, DFF, HIDDEN), lambda i: (0, 0, 0)),
            pl.BlockSpec((HIDDEN, SHARED_INTER), lambda i: (0, 0)),
            pl.BlockSpec((SHARED_INTER, HIDDEN), lambda i: (0, 0)),
        ],
        out_specs=pl.BlockSpec((TB, HIDDEN), lambda i: (i, 0)),
        out_shape=jax.ShapeDtypeStruct((TOKENS, HIDDEN), jnp.float32),
    )(x, gate_weight, e_score_correction_bias, w1b, w2b, sw1b, sw2b)
    return out.reshape(orig_shape)


# speedup vs baseline: 1.2191x; 1.2191x over previous
import jax
import jax.numpy as jnp
from jax import lax
from jax.experimental import pallas as pl
from jax.experimental.pallas import tpu as pltpu

TOKENS = 2048
HIDDEN = 1024
E = 8
TOPK = 2
NGROUP = 4
EG = E // NGROUP
DFF = 512
SHARED_INTER = 1024
RSF = 2.5

TB = 256


def _relu2(x):
    return jnp.square(jnp.maximum(x, 0.0))


def _moe_block(x_ref, gw_ref, bias_ref, w1_ref, w2_ref, sw1_ref, sw2_ref, out_ref):
    x = x_ref[...]

    logits = jnp.dot(x, gw_ref[...].T, preferred_element_type=jnp.float32)
    scores = jax.nn.sigmoid(logits)
    swb = scores + bias_ref[...]

    gs = swb.reshape(TB, NGROUP, EG).sum(axis=-1)
    gidx = lax.broadcasted_iota(jnp.int32, (TB, NGROUP), 1)
    g1 = jnp.argmax(gs, axis=1)
    gs2 = jnp.where(gidx == g1[:, None], -jnp.inf, gs)
    g2 = jnp.argmax(gs2, axis=1)

    eidx = lax.broadcasted_iota(jnp.int32, (TB, E), 1)
    egrp = eidx // EG
    emask = (egrp == g1[:, None]) | (egrp == g2[:, None])
    masked = jnp.where(emask, swb, -jnp.inf)
    e1 = jnp.argmax(masked, axis=1)
    m2 = jnp.where(eidx == e1[:, None], -jnp.inf, masked)
    e2 = jnp.argmax(m2, axis=1)
    oh1 = (eidx == e1[:, None]).astype(jnp.float32)
    oh2 = (eidx == e2[:, None]).astype(jnp.float32)
    s1 = jnp.sum(oh1 * scores, axis=1)
    s2 = jnp.sum(oh2 * scores, axis=1)
    rn = RSF / (s1 + s2 + 1e-20)
    gates = oh1 * (s1 * rn)[:, None] + oh2 * (s2 * rn)[:, None]

    xb = x.astype(jnp.bfloat16)

    h = _relu2(jnp.dot(xb, sw1_ref[...].astype(jnp.bfloat16), preferred_element_type=jnp.float32))
    acc = jnp.dot(h.astype(jnp.bfloat16), sw2_ref[...].astype(jnp.bfloat16),
                  preferred_element_type=jnp.float32)

    for e in range(E):
        he = _relu2(jnp.dot(xb, w1_ref[e].astype(jnp.bfloat16), preferred_element_type=jnp.float32))
        ye = jnp.dot(he.astype(jnp.bfloat16), w2_ref[e].astype(jnp.bfloat16),
                     preferred_element_type=jnp.float32)
        acc = acc + gates[:, e:e + 1] * ye

    out_ref[...] = acc


def kernel(hidden_states, gate_weight, e_score_correction_bias, w1, w2, shared_w1, shared_w2):
    orig_shape = hidden_states.shape
    x = hidden_states.reshape(-1, HIDDEN)

    grid = (TOKENS // TB,)
    out = pl.pallas_call(
        _moe_block,
        grid=grid,
        in_specs=[
            pl.BlockSpec((TB, HIDDEN), lambda i: (i, 0)),
            pl.BlockSpec((E, HIDDEN), lambda i: (0, 0)),
            pl.BlockSpec((E,), lambda i: (0,)),
            pl.BlockSpec((E, HIDDEN, DFF), lambda i: (0, 0, 0)),
            pl.BlockSpec((E, DFF, HIDDEN), lambda i: (0, 0, 0)),
            pl.BlockSpec((HIDDEN, SHARED_INTER), lambda i: (0, 0)),
            pl.BlockSpec((SHARED_INTER, HIDDEN), lambda i: (0, 0)),
        ],
        out_specs=pl.BlockSpec((TB, HIDDEN), lambda i: (i, 0)),
        out_shape=jax.ShapeDtypeStruct((TOKENS, HIDDEN), jnp.float32),
    )(x, gate_weight, e_score_correction_bias, w1, w2, shared_w1, shared_w2)
    return out.reshape(orig_shape)


# speedup vs baseline: 1.4180x; 1.1631x over previous
import jax
import jax.numpy as jnp
from jax import lax
from jax.experimental import pallas as pl
from jax.experimental.pallas import tpu as pltpu

TOKENS = 2048
HIDDEN = 1024
E = 8
TOPK = 2
NGROUP = 4
EG = E // NGROUP
DFF = 512
SHARED_INTER = 1024
RSF = 2.5

TB = 256


def _relu2(x):
    return jnp.square(jnp.maximum(x, 0.0))


def _moe_block(x_ref, gw_ref, bias_ref, w1_ref, w2_ref, sw1_ref, sw2_ref, out_ref):
    x = x_ref[...]

    logits = jnp.dot(x, gw_ref[...].T, preferred_element_type=jnp.float32)
    scores = jax.nn.sigmoid(logits)
    swb = scores + bias_ref[...]

    gs = swb.reshape(TB, NGROUP, EG).sum(axis=-1)
    gidx = lax.broadcasted_iota(jnp.int32, (TB, NGROUP), 1)
    g1 = jnp.argmax(gs, axis=1)
    gs2 = jnp.where(gidx == g1[:, None], -jnp.inf, gs)
    g2 = jnp.argmax(gs2, axis=1)

    eidx = lax.broadcasted_iota(jnp.int32, (TB, E), 1)
    egrp = eidx // EG
    emask = (egrp == g1[:, None]) | (egrp == g2[:, None])
    masked = jnp.where(emask, swb, -jnp.inf)
    e1 = jnp.argmax(masked, axis=1)
    m2 = jnp.where(eidx == e1[:, None], -jnp.inf, masked)
    e2 = jnp.argmax(m2, axis=1)
    oh1 = (eidx == e1[:, None]).astype(jnp.float32)
    oh2 = (eidx == e2[:, None]).astype(jnp.float32)
    s1 = jnp.sum(oh1 * scores, axis=1)
    s2 = jnp.sum(oh2 * scores, axis=1)
    rn = RSF / (s1 + s2 + 1e-20)
    gates = oh1 * (s1 * rn)[:, None] + oh2 * (s2 * rn)[:, None]

    xb = x.astype(jnp.bfloat16)

    h = _relu2(jnp.dot(xb, sw1_ref[...].astype(jnp.bfloat16), preferred_element_type=jnp.float32))
    acc = jnp.dot(h.astype(jnp.bfloat16), sw2_ref[...].astype(jnp.bfloat16),
                  preferred_element_type=jnp.float32)

    hs = []
    for e in range(E):
        he = _relu2(jnp.dot(xb, w1_ref[e].astype(jnp.bfloat16),
                            preferred_element_type=jnp.float32))
        hs.append((gates[:, e:e + 1] * he).astype(jnp.bfloat16))
    hall = jnp.concatenate(hs, axis=1)
    acc = acc + jnp.dot(hall, w2_ref[...].astype(jnp.bfloat16),
                        preferred_element_type=jnp.float32)

    out_ref[...] = acc


def kernel(hidden_states, gate_weight, e_score_correction_bias, w1, w2, shared_w1, shared_w2):
    orig_shape = hidden_states.shape
    x = hidden_states.reshape(-1, HIDDEN)

    grid = (TOKENS // TB,)
    out = pl.pallas_call(
        _moe_block,
        grid=grid,
        in_specs=[
            pl.BlockSpec((TB, HIDDEN), lambda i: (i, 0)),
            pl.BlockSpec((E, HIDDEN), lambda i: (0, 0)),
            pl.BlockSpec((E,), lambda i: (0,)),
            pl.BlockSpec((E, HIDDEN, DFF), lambda i: (0, 0, 0)),
            pl.BlockSpec((E * DFF, HIDDEN), lambda i: (0, 0)),
            pl.BlockSpec((HIDDEN, SHARED_INTER), lambda i: (0, 0)),
            pl.BlockSpec((SHARED_INTER, HIDDEN), lambda i: (0, 0)),
        ],
        out_specs=pl.BlockSpec((TB, HIDDEN), lambda i: (i, 0)),
        out_shape=jax.ShapeDtypeStruct((TOKENS, HIDDEN), jnp.float32),
    )(x, gate_weight, e_score_correction_bias, w1, w2.reshape(E * DFF, HIDDEN), shared_w1, shared_w2)
    return out.reshape(orig_shape)


# speedup vs baseline: 1.4411x; 1.0163x over previous
import jax
import jax.numpy as jnp
from jax import lax
from jax.experimental import pallas as pl
from jax.experimental.pallas import tpu as pltpu

TOKENS = 2048
HIDDEN = 1024
E = 8
TOPK = 2
NGROUP = 4
EG = E // NGROUP
DFF = 512
SHARED_INTER = 1024
RSF = 2.5

TB = 512


def _relu2(x):
    return jnp.square(jnp.maximum(x, 0.0))


def _moe_block(x_ref, gw_ref, bias_ref, w1_ref, w2_ref, sw1_ref, sw2_ref, out_ref):
    x = x_ref[...]

    logits = jnp.dot(x, gw_ref[...].T, preferred_element_type=jnp.float32)
    scores = jax.nn.sigmoid(logits)
    swb = scores + bias_ref[...]

    gs = swb.reshape(TB, NGROUP, EG).sum(axis=-1)
    gidx = lax.broadcasted_iota(jnp.int32, (TB, NGROUP), 1)
    g1 = jnp.argmax(gs, axis=1)
    gs2 = jnp.where(gidx == g1[:, None], -jnp.inf, gs)
    g2 = jnp.argmax(gs2, axis=1)

    eidx = lax.broadcasted_iota(jnp.int32, (TB, E), 1)
    egrp = eidx // EG
    emask = (egrp == g1[:, None]) | (egrp == g2[:, None])
    masked = jnp.where(emask, swb, -jnp.inf)
    e1 = jnp.argmax(masked, axis=1)
    m2 = jnp.where(eidx == e1[:, None], -jnp.inf, masked)
    e2 = jnp.argmax(m2, axis=1)
    oh1 = (eidx == e1[:, None]).astype(jnp.float32)
    oh2 = (eidx == e2[:, None]).astype(jnp.float32)
    s1 = jnp.sum(oh1 * scores, axis=1)
    s2 = jnp.sum(oh2 * scores, axis=1)
    rn = RSF / (s1 + s2 + 1e-20)
    gates = oh1 * (s1 * rn)[:, None] + oh2 * (s2 * rn)[:, None]

    xb = x.astype(jnp.bfloat16)

    h = _relu2(jnp.dot(xb, sw1_ref[...].astype(jnp.bfloat16), preferred_element_type=jnp.float32))
    acc = jnp.dot(h.astype(jnp.bfloat16), sw2_ref[...].astype(jnp.bfloat16),
                  preferred_element_type=jnp.float32)

    hs = []
    for e in range(E):
        he = _relu2(jnp.dot(xb, w1_ref[e].astype(jnp.bfloat16),
                            preferred_element_type=jnp.float32))
        hs.append((gates[:, e:e + 1] * he).astype(jnp.bfloat16))
    hall = jnp.concatenate(hs, axis=1)
    acc = acc + jnp.dot(hall, w2_ref[...].astype(jnp.bfloat16),
                        preferred_element_type=jnp.float32)

    out_ref[...] = acc


def kernel(hidden_states, gate_weight, e_score_correction_bias, w1, w2, shared_w1, shared_w2):
    orig_shape = hidden_states.shape
    x = hidden_states.reshape(-1, HIDDEN)

    grid = (TOKENS // TB,)
    out = pl.pallas_call(
        _moe_block,
        grid=grid,
        in_specs=[
            pl.BlockSpec((TB, HIDDEN), lambda i: (i, 0)),
            pl.BlockSpec((E, HIDDEN), lambda i: (0, 0)),
            pl.BlockSpec((E,), lambda i: (0,)),
            pl.BlockSpec((E, HIDDEN, DFF), lambda i: (0, 0, 0)),
            pl.BlockSpec((E * DFF, HIDDEN), lambda i: (0, 0)),
            pl.BlockSpec((HIDDEN, SHARED_INTER), lambda i: (0, 0)),
            pl.BlockSpec((SHARED_INTER, HIDDEN), lambda i: (0, 0)),
        ],
        out_specs=pl.BlockSpec((TB, HIDDEN), lambda i: (i, 0)),
        out_shape=jax.ShapeDtypeStruct((TOKENS, HIDDEN), jnp.float32),
    )(x, gate_weight, e_score_correction_bias, w1, w2.reshape(E * DFF, HIDDEN), shared_w1, shared_w2)
    return out.reshape(orig_shape)
